# baseline (speedup 1.0000x reference)
import jax
import jax.numpy as jnp
from jax.experimental import pallas as pl
from jax.experimental.pallas import tpu as pltpu

_TOP_K = 2
_FT = 512


def _moe_body(gidx_ref, iirow_ref, ssr_ref, ssc_ref, x_ref, w1_ref, b1_ref,
              w2_ref, b2_ref, out_ref, xe_ref, acc_ref):
    e = pl.program_id(0)
    f = pl.program_id(1)
    nf = pl.num_programs(1)
    T = x_ref.shape[0]
    cap = xe_ref.shape[0]

    @pl.when(jnp.logical_and(e == 0, f == 0))
    def _():
        out_ref[...] = jnp.zeros_like(out_ref)

    @pl.when(f == 0)
    def _():
        g = gidx_ref[0]
        tcol = jax.lax.broadcasted_iota(jnp.int32, (cap, T), 1)
        onehot = jnp.where(tcol == g, 1.0, 0.0).astype(jnp.bfloat16)
        xe_ref[...] = jnp.dot(onehot, x_ref[...],
                              preferred_element_type=jnp.float32
                              ).astype(jnp.bfloat16)
        acc_ref[...] = jnp.broadcast_to(b2_ref[0], acc_ref.shape)

    w1b = w1_ref[0].astype(jnp.bfloat16)
    w2b = w2_ref[0].astype(jnp.bfloat16)
    h = jnp.maximum(
        jnp.dot(xe_ref[...], w1b, preferred_element_type=jnp.float32)
        + b1_ref[0], 0.0)
    acc_ref[...] += jnp.dot(h.astype(jnp.bfloat16), w2b,
                            preferred_element_type=jnp.float32)

    @pl.when(f == nf - 1)
    def _():
        ssr = ssr_ref[0]
        ssc = ssc_ref[0]
        iir = iirow_ref[0]
        iop = jax.lax.broadcasted_iota(jnp.int32, (cap, cap), 0)
        ioq = jax.lax.broadcasted_iota(jnp.int32, (cap, cap), 1)
        before = jnp.logical_or(
            ssc > ssr, jnp.logical_and(ssc == ssr, iop < ioq))
        invp = jnp.sum(jnp.where(before, 1, 0), axis=0, keepdims=True)
        bm = jnp.where(iop == invp, ssr, 0.0)
        y2 = jnp.dot(bm, acc_ref[...], preferred_element_type=jnp.float32)
        trow = jax.lax.broadcasted_iota(jnp.int32, (T, cap), 0)
        am = jnp.where(trow == iir, 1.0, 0.0).astype(jnp.bfloat16)
        out_ref[...] += jnp.dot(am, y2.astype(jnp.bfloat16),
                                preferred_element_type=jnp.float32)


def kernel(x, Wg, bg, W1, b1, W2, b2):
    B, S, H = x.shape
    T = B * S
    E = Wg.shape[-1]
    F = W1.shape[-1]
    K = _TOP_K
    cap = max(T * K // E, 1)
    nf = F // _FT
    xf = x.reshape(T, H)

    gate_logits = xf @ Wg + bg

    i1 = jnp.argmax(gate_logits, axis=-1).astype(jnp.int32)
    v1 = jnp.max(gate_logits, axis=-1)
    cols = jnp.arange(E, dtype=jnp.int32)[None, :]
    masked = jnp.where(cols == i1[:, None], -jnp.inf, gate_logits)
    i2 = jnp.argmax(masked, axis=-1).astype(jnp.int32)
    v2 = jnp.max(masked, axis=-1)
    scores = jnp.stack([v1, v2], axis=-1)
    eidx = jnp.stack([i1, i2], axis=-1)
    sc = jax.nn.softmax(scores, axis=-1)

    p_full = jax.nn.softmax(gate_logits, axis=-1)
    m_i = jnp.mean(p_full, axis=0)
    ohk = (cols[None] == eidx[:, :, None]).astype(jnp.float32)
    f_i = jnp.mean(ohk, axis=(0, 1))
    aux = 0.01 * jnp.sum(f_i * m_i) / E

    a = eidx.reshape(-1)
    ohi = (cols == a[:, None]).astype(jnp.int32)
    rank = jnp.sum((jnp.cumsum(ohi, axis=0) - ohi) * ohi, axis=1)
    valid = rank < cap
    slot = jnp.where(valid, a * cap + rank, E * cap)
    tok = (jnp.arange(T * K, dtype=jnp.int32) // K).astype(jnp.float32)
    packed = jnp.stack([tok + 1.0, sc.reshape(-1)], axis=-1)
    dense = jnp.zeros((E * cap + 1, 2), jnp.float32).at[slot].set(packed)
    ii = dense[:E * cap, 0].astype(jnp.int32).reshape(E, cap) - 1
    ss = dense[:E * cap, 1].reshape(E, cap)
    gidx = jnp.maximum(ii, 0)

    out = pl.pallas_call(
        _moe_body,
        grid=(E, nf),
        in_specs=[
            pl.BlockSpec((1, cap, 1), lambda e, f: (e, 0, 0)),
            pl.BlockSpec((1, 1, cap), lambda e, f: (e, 0, 0)),
            pl.BlockSpec((1, 1, cap), lambda e, f: (e, 0, 0)),
            pl.BlockSpec((1, cap, 1), lambda e, f: (e, 0, 0)),
            pl.BlockSpec((T, H), lambda e, f: (0, 0)),
            pl.BlockSpec((1, H, _FT), lambda e, f: (e, 0, f)),
            pl.BlockSpec((1, 1, _FT), lambda e, f: (e, 0, f)),
            pl.BlockSpec((1, _FT, H), lambda e, f: (e, f, 0)),
            pl.BlockSpec((1, 1, H), lambda e, f: (e, 0, 0)),
        ],
        out_specs=pl.BlockSpec((T, H), lambda e, f: (0, 0)),
        out_shape=jax.ShapeDtypeStruct((T, H), jnp.float32),
        scratch_shapes=[
            pltpu.VMEM((cap, H), jnp.bfloat16),
            pltpu.VMEM((cap, H), jnp.float32),
        ],
        compiler_params=pltpu.CompilerParams(
            dimension_semantics=("arbitrary", "arbitrary")),
    )(gidx.reshape(E, cap, 1), ii.reshape(E, 1, cap), ss.reshape(E, 1, cap),
      ss.reshape(E, cap, 1), xf.astype(jnp.bfloat16), W1,
      b1.reshape(E, 1, F), W2, b2.reshape(E, 1, H))

    return out.reshape(B, S, H), aux


# speedup vs baseline: 1.1439x; 1.1439x over previous
import jax
import jax.numpy as jnp
from jax.experimental import pallas as pl
from jax.experimental.pallas import tpu as pltpu

_TOP_K = 2
_FT = 1024


def _moe_body(gidx_ref, iirow_ref, ssr_ref, ssc_ref, x_ref, w1_ref, b1_ref,
              w2_ref, b2_ref, out_ref, xe_ref, acc_ref):
    e = pl.program_id(0)
    f = pl.program_id(1)
    nf = pl.num_programs(1)
    T = x_ref.shape[0]
    cap = xe_ref.shape[0]

    @pl.when(jnp.logical_and(e == 0, f == 0))
    def _():
        out_ref[...] = jnp.zeros_like(out_ref)

    @pl.when(f == 0)
    def _():
        g = gidx_ref[0]
        tcol = jax.lax.broadcasted_iota(jnp.int32, (cap, T), 1)
        onehot = jnp.where(tcol == g, 1.0, 0.0)
        xe_ref[...] = jnp.dot(onehot, x_ref[...],
                              preferred_element_type=jnp.float32)
        acc_ref[...] = jnp.broadcast_to(b2_ref[0], acc_ref.shape)

    h = jnp.maximum(
        jnp.dot(xe_ref[...], w1_ref[0], preferred_element_type=jnp.float32)
        + b1_ref[0], 0.0)
    acc_ref[...] += jnp.dot(h, w2_ref[0], preferred_element_type=jnp.float32)

    @pl.when(f == nf - 1)
    def _():
        ssr = ssr_ref[0]
        ssc = ssc_ref[0]
        iir = iirow_ref[0]
        iop = jax.lax.broadcasted_iota(jnp.int32, (cap, cap), 0)
        ioq = jax.lax.broadcasted_iota(jnp.int32, (cap, cap), 1)
        before = jnp.logical_or(
            ssc > ssr, jnp.logical_and(ssc == ssr, iop < ioq))
        invp = jnp.sum(jnp.where(before, 1, 0), axis=0, keepdims=True)
        bm = jnp.where(iop == invp, ssr, 0.0)
        y2 = jnp.dot(bm, acc_ref[...], preferred_element_type=jnp.float32)
        trow = jax.lax.broadcasted_iota(jnp.int32, (T, cap), 0)
        am = jnp.where(trow == iir, 1.0, 0.0)
        out_ref[...] += jnp.dot(am, y2, preferred_element_type=jnp.float32)


def kernel(x, Wg, bg, W1, b1, W2, b2):
    B, S, H = x.shape
    T = B * S
    E = Wg.shape[-1]
    F = W1.shape[-1]
    K = _TOP_K
    cap = max(T * K // E, 1)
    nf = F // _FT
    xf = x.reshape(T, H)

    gate_logits = xf @ Wg + bg

    i1 = jnp.argmax(gate_logits, axis=-1).astype(jnp.int32)
    v1 = jnp.max(gate_logits, axis=-1)
    cols = jnp.arange(E, dtype=jnp.int32)[None, :]
    masked = jnp.where(cols == i1[:, None], -jnp.inf, gate_logits)
    i2 = jnp.argmax(masked, axis=-1).astype(jnp.int32)
    v2 = jnp.max(masked, axis=-1)
    scores = jnp.stack([v1, v2], axis=-1)
    eidx = jnp.stack([i1, i2], axis=-1)
    sc = jax.nn.softmax(scores, axis=-1)

    p_full = jax.nn.softmax(gate_logits, axis=-1)
    m_i = jnp.mean(p_full, axis=0)
    ohk = (cols[None] == eidx[:, :, None]).astype(jnp.float32)
    f_i = jnp.mean(ohk, axis=(0, 1))
    aux = 0.01 * jnp.sum(f_i * m_i) / E

    a = eidx.reshape(-1)
    ohi = (cols == a[:, None]).astype(jnp.int32)
    rank = jnp.sum((jnp.cumsum(ohi, axis=0) - ohi) * ohi, axis=1)
    valid = rank < cap
    slot = jnp.where(valid, a * cap + rank, E * cap)
    tok = (jnp.arange(T * K, dtype=jnp.int32) // K).astype(jnp.float32)
    packed = jnp.stack([tok + 1.0, sc.reshape(-1)], axis=-1)
    dense = jnp.zeros((E * cap + 1, 2), jnp.float32).at[slot].set(packed)
    ii = dense[:E * cap, 0].astype(jnp.int32).reshape(E, cap) - 1
    ss = dense[:E * cap, 1].reshape(E, cap)
    gidx = jnp.maximum(ii, 0)

    out = pl.pallas_call(
        _moe_body,
        grid=(E, nf),
        in_specs=[
            pl.BlockSpec((1, cap, 1), lambda e, f: (e, 0, 0)),
            pl.BlockSpec((1, 1, cap), lambda e, f: (e, 0, 0)),
            pl.BlockSpec((1, 1, cap), lambda e, f: (e, 0, 0)),
            pl.BlockSpec((1, cap, 1), lambda e, f: (e, 0, 0)),
            pl.BlockSpec((T, H), lambda e, f: (0, 0)),
            pl.BlockSpec((1, H, _FT), lambda e, f: (e, 0, f)),
            pl.BlockSpec((1, 1, _FT), lambda e, f: (e, 0, f)),
            pl.BlockSpec((1, _FT, H), lambda e, f: (e, f, 0)),
            pl.BlockSpec((1, 1, H), lambda e, f: (e, 0, 0)),
        ],
        out_specs=pl.BlockSpec((T, H), lambda e, f: (0, 0)),
        out_shape=jax.ShapeDtypeStruct((T, H), jnp.float32),
        scratch_shapes=[
            pltpu.VMEM((cap, H), jnp.float32),
            pltpu.VMEM((cap, H), jnp.float32),
        ],
        compiler_params=pltpu.CompilerParams(
            dimension_semantics=("arbitrary", "arbitrary")),
    )(gidx.reshape(E, cap, 1), ii.reshape(E, 1, cap), ss.reshape(E, 1, cap),
      ss.reshape(E, cap, 1), xf, W1,
      b1.reshape(E, 1, F), W2, b2.reshape(E, 1, H))

    return out.reshape(B, S, H), aux
